# baseline (speedup 1.0000x reference)
import functools

import jax
import jax.numpy as jnp
from jax import lax
from jax.experimental import pallas as pl
from jax.experimental.pallas import tpu as pltpu
from jax.experimental.pallas import tpu_sc as plsc

_PADDING_IDX = 1
_L = 16
_C = 32
_NBUF = 3


def _sc_lookup(total_rows, seq_len, embed_dim):
    nw = 32
    rows_w = total_rows // nw
    nchunk = rows_w // _C
    ngroups = rows_w // _L
    idx_pad = rows_w + 2 * _C

    mesh = plsc.VectorSubcoreMesh(core_axis_name="c", subcore_axis_name="s")

    @functools.partial(
        pl.kernel, mesh=mesh,
        out_type=jax.ShapeDtypeStruct((total_rows, embed_dim), jnp.float32),
        scratch_types=[
            pltpu.VMEM((idx_pad,), jnp.int32),
            pltpu.VMEM((rows_w,), jnp.int32),
            pltpu.VMEM((_NBUF, _C, embed_dim), jnp.float32),
            pltpu.SemaphoreType.DMA,
            pltpu.SemaphoreType.DMA,
            pltpu.SemaphoreType.DMA,
            pltpu.SemaphoreType.DMA,
            pltpu.SemaphoreType.DMA,
            pltpu.SemaphoreType.DMA,
        ],
    )
    def k(table_hbm, x_hbm, out_hbm, idx_v, x_v, rows_v,
          g0, g1, g2, s0_, s1_, s2_):
        gsems = (g0, g1, g2)
        ssems = (s0_, s1_, s2_)
        wid = lax.axis_index("s") * 2 + lax.axis_index("c")
        row0 = wid * rows_w
        seq0 = lax.rem(row0, seq_len)
        pltpu.sync_copy(x_hbm.at[pl.ds(row0, rows_w)], x_v)

        lane = lax.iota(jnp.int32, _L)

        def mk_idx(g, _):
            xv = x_v[pl.ds(g * _L, _L)]
            pos = (seq0 + _PADDING_IDX + 1 + g * _L) + lane
            idx_v[pl.ds(g * _L, _L)] = jnp.where(
                xv == _PADDING_IDX, _PADDING_IDX, pos)
            return 0

        lax.fori_loop(0, ngroups, mk_idx, 0)
        zero = jnp.zeros((_L,), jnp.int32)
        for g in range(ngroups, idx_pad // _L):
            idx_v[pl.ds(g * _L, _L)] = zero

        def gather(kk, b):
            return pltpu.make_async_copy(
                table_hbm.at[idx_v.at[pl.ds(kk * _C, _C)]],
                rows_v.at[b], gsems[b])

        def scatter(kk, b):
            return pltpu.make_async_copy(
                rows_v.at[b], out_hbm.at[pl.ds(row0 + kk * _C, _C)],
                ssems[b])

        gather(0, 0).start()

        def step(g, _):
            for b0 in range(_NBUF):
                kk = g * _NBUF + b0
                bn = (b0 + 1) % _NBUF

                @pl.when(kk >= 2)
                def _():
                    scatter(kk - 2, bn).wait()
                gather(kk + 1, bn).start()
                gather(kk, b0).wait()
                scatter(kk, b0).start()
            return 0

        lax.fori_loop(0, nchunk // _NBUF, step, 0)
        for kk in range((nchunk // _NBUF) * _NBUF, nchunk):
            b0 = kk % _NBUF
            bn = (b0 + 1) % _NBUF
            scatter(kk - 2, bn).wait()
            gather(kk + 1, bn).start()
            gather(kk, b0).wait()
            scatter(kk, b0).start()
        gather(nchunk, nchunk % _NBUF).wait()
        scatter(nchunk - 2, (nchunk - 2) % _NBUF).wait()
        scatter(nchunk - 1, (nchunk - 1) % _NBUF).wait()

    return k


def kernel(x, weights):
    bsz, seq_len = x.shape
    embed_dim = weights.shape[1]
    total = bsz * seq_len
    flat = _sc_lookup(total, seq_len, embed_dim)(weights, x.reshape(-1))
    return jax.lax.stop_gradient(flat.reshape(bsz, seq_len, embed_dim))
